# baseline (speedup 1.0000x reference)
import functools

import jax
import jax.numpy as jnp
from jax import lax
from jax.experimental import pallas as pl
from jax.experimental.pallas import tpu as pltpu
from jax.experimental.pallas import tpu_sc as plsc

_B = 4
_S = 8192
_F = 1024
_NC = 2
_NS = 16
_NW = _NC * _NS
_RPW = _S // _NW
_R = 16
_NBLK = _RPW // _R
_NSET = 3
_NSTEP = _NBLK * _B


def _sc_body(in_hbm, pe_hbm, out_hbm, *scratch):
    pe_v = list(scratch[0:2])
    in_v = list(scratch[2:2 + _NSET])
    spe = scratch[2 + _NSET]
    sin = list(scratch[3 + _NSET:3 + 2 * _NSET])
    sout = list(scratch[3 + 2 * _NSET:3 + 3 * _NSET])

    wid = lax.axis_index("s") * _NC + lax.axis_index("c")
    base = wid * _RPW

    def issue_pe(blk):
        r0 = base + blk * _R
        return pltpu.async_copy(
            pe_hbm.at[pl.ds(r0, _R)], pe_v[blk % 2], spe)

    def issue_in(step, p):
        blk, b = divmod(step, _B)
        r0 = base + blk * _R
        return pltpu.async_copy(
            in_hbm.at[pl.ds(b * _S + r0, _R)], in_v[p], sin[p])

    def issue_out(step, p):
        blk, b = divmod(step, _B)
        r0 = base + blk * _R
        return pltpu.async_copy(
            in_v[p], out_hbm.at[pl.ds(b * _S + r0, _R)], sout[p])

    hin = [None] * _NSET
    hout = [None] * _NSET
    hpe = [None, None]
    hpe[0] = issue_pe(0)
    hin[0] = issue_in(0, 0)
    hin[1] = issue_in(1, 1)

    for step in range(_NSTEP):
        blk, b = divmod(step, _B)
        p = step % _NSET
        if b == 0:
            hpe[blk % 2].wait()
            if blk + 1 < _NBLK:
                hpe[(blk + 1) % 2] = issue_pe(blk + 1)
        hin[p].wait()
        nxt = step + 2
        if nxt < _NSTEP:
            p2 = nxt % _NSET
            if hout[p2] is not None:
                hout[p2].wait()
                hout[p2] = None
            hin[p2] = issue_in(nxt, p2)

        def chunk(j, carry, p=p, m=blk % 2):
            o = j * 16
            for r in range(_R):
                in_v[p][r, pl.ds(o, 16)] = (
                    in_v[p][r, pl.ds(o, 16)] + pe_v[m][r, pl.ds(o, 16)]
                )
            return carry

        lax.fori_loop(0, _F // 16, chunk, 0)
        hout[p] = issue_out(step, p)

    for h in hout:
        if h is not None:
            h.wait()


_sc_add = functools.partial(
    pl.kernel,
    out_type=jax.ShapeDtypeStruct((_B * _S, _F), jnp.float32),
    mesh=plsc.VectorSubcoreMesh(core_axis_name="c", subcore_axis_name="s"),
    compiler_params=pltpu.CompilerParams(use_tc_tiling_on_sc=True),
    scratch_types=(
        [pltpu.VMEM((_R, _F), jnp.float32) for _ in range(2)]
        + [pltpu.VMEM((_R, _F), jnp.float32) for _ in range(_NSET)]
        + [pltpu.SemaphoreType.DMA for _ in range(1 + 2 * _NSET)]
    ),
)(_sc_body)


def kernel(inputs, positional_encoding):
    b, s, f = inputs.shape
    out = _sc_add(inputs.reshape(b * s, f), positional_encoding[:s])
    return out.reshape(b, s, f)


# speedup vs baseline: 1.4235x; 1.4235x over previous
import functools

import jax
import jax.numpy as jnp
from jax import lax
from jax.experimental import pallas as pl
from jax.experimental.pallas import tpu as pltpu
from jax.experimental.pallas import tpu_sc as plsc

_B = 4
_S = 8192
_F = 1024
_NC = 2
_NS = 16
_NW = _NC * _NS
_RPW = _S // _NW
_R = 8
_NBLK = _RPW // _R
_NSET = 3


def _sc_body(in_hbm, pe_hbm, out_hbm, *scratch):
    pe_v = list(scratch[0:_NSET])
    in_v = list(scratch[_NSET:2 * _NSET])
    sin = list(scratch[2 * _NSET:3 * _NSET])
    sout = list(scratch[3 * _NSET:4 * _NSET])

    wid = lax.axis_index("s") * _NC + lax.axis_index("c")
    base = wid * _RPW

    def issue_in(i, p):
        r0 = base + i * _R
        hs = [pltpu.async_copy(pe_hbm.at[pl.ds(r0, _R)], pe_v[p], sin[p])]
        for b in range(_B):
            hs.append(pltpu.async_copy(
                in_hbm.at[pl.ds(b * _S + r0, _R)],
                in_v[p].at[pl.ds(b * _R, _R)],
                sin[p],
            ))
        return hs

    def issue_out(i, p):
        r0 = base + i * _R
        return [pltpu.async_copy(
            in_v[p].at[pl.ds(b * _R, _R)],
            out_hbm.at[pl.ds(b * _S + r0, _R)],
            sout[p],
        ) for b in range(_B)]

    hin = [None] * _NSET
    hout = [None] * _NSET
    hin[0] = issue_in(0, 0)
    hin[1] = issue_in(1, 1)

    for i in range(_NBLK):
        p = i % _NSET
        nxt = i + 2
        if nxt < _NBLK:
            p2 = nxt % _NSET
            if hout[p2] is not None:
                for h in hout[p2]:
                    h.wait()
                hout[p2] = None
            hin[p2] = issue_in(nxt, p2)
        for h in hin[p]:
            h.wait()

        def chunk(j, carry, p=p):
            o = j * 16
            for r in range(_R):
                pv = pe_v[p][r, pl.ds(o, 16)]
                for b in range(_B):
                    in_v[p][b * _R + r, pl.ds(o, 16)] = (
                        in_v[p][b * _R + r, pl.ds(o, 16)] + pv
                    )
            return carry

        lax.fori_loop(0, _F // 16, chunk, 0)
        hout[p] = issue_out(i, p)

    for hs in hout:
        if hs is not None:
            for h in hs:
                h.wait()


_sc_add = functools.partial(
    pl.kernel,
    out_type=jax.ShapeDtypeStruct((_B * _S, _F), jnp.float32),
    mesh=plsc.VectorSubcoreMesh(core_axis_name="c", subcore_axis_name="s"),
    compiler_params=pltpu.CompilerParams(use_tc_tiling_on_sc=True),
    scratch_types=(
        [pltpu.VMEM((_R, _F), jnp.float32) for _ in range(_NSET)]
        + [pltpu.VMEM((_B * _R, _F), jnp.float32) for _ in range(_NSET)]
        + [pltpu.SemaphoreType.DMA for _ in range(2 * _NSET)]
    ),
)(_sc_body)


def kernel(inputs, positional_encoding):
    b, s, f = inputs.shape
    out = _sc_add(inputs.reshape(b * s, f), positional_encoding[:s])
    return out.reshape(b, s, f)


# speedup vs baseline: 1.4668x; 1.0304x over previous
import functools

import jax
import jax.numpy as jnp
from jax import lax
from jax.experimental import pallas as pl
from jax.experimental.pallas import tpu as pltpu
from jax.experimental.pallas import tpu_sc as plsc

_B = 4
_S = 8192
_F = 1024
_NC = 2
_NS = 16
_NW = _NC * _NS
_RPW = _S // _NW
_R = 8
_NBLK = _RPW // _R
_NSET = 3


def _sc_body(in_hbm, pe_hbm, out_hbm, *scratch):
    pe_v = list(scratch[0:_NSET])
    in_v = list(scratch[_NSET:2 * _NSET])
    sin = list(scratch[2 * _NSET:3 * _NSET])
    sout = list(scratch[3 * _NSET:4 * _NSET])

    wid = lax.axis_index("s") * _NC + lax.axis_index("c")
    base = wid * _RPW

    def issue_in(i, p):
        r0 = base + i * _R
        return [
            pltpu.async_copy(pe_hbm.at[pl.ds(r0, _R)], pe_v[p], sin[p]),
            pltpu.async_copy(in_hbm.at[:, pl.ds(r0, _R)], in_v[p], sin[p]),
        ]

    def issue_out(i, p):
        r0 = base + i * _R
        return [pltpu.async_copy(
            in_v[p], out_hbm.at[:, pl.ds(r0, _R)], sout[p])]

    hin = [None] * _NSET
    hout = [None] * _NSET
    hin[0] = issue_in(0, 0)
    hin[1] = issue_in(1, 1)

    for i in range(_NBLK):
        p = i % _NSET
        for h in hin[p]:
            h.wait()
        nxt = i + 2
        if nxt < _NBLK:
            p2 = nxt % _NSET
            if hout[p2] is not None:
                for h in hout[p2]:
                    h.wait()
                hout[p2] = None
            hin[p2] = issue_in(nxt, p2)

        def chunk(j, carry, p=p):
            o = j * 16
            for r in range(_R):
                pv = pe_v[p][r, pl.ds(o, 16)]
                for b in range(_B):
                    in_v[p][b, r, pl.ds(o, 16)] = (
                        in_v[p][b, r, pl.ds(o, 16)] + pv
                    )
            return carry

        lax.fori_loop(0, _F // 16, chunk, 0)
        hout[p] = issue_out(i, p)

    for hs in hout:
        if hs is not None:
            for h in hs:
                h.wait()


_sc_add = functools.partial(
    pl.kernel,
    out_type=jax.ShapeDtypeStruct((_B, _S, _F), jnp.float32),
    mesh=plsc.VectorSubcoreMesh(core_axis_name="c", subcore_axis_name="s"),
    compiler_params=pltpu.CompilerParams(use_tc_tiling_on_sc=True),
    scratch_types=(
        [pltpu.VMEM((_R, _F), jnp.float32) for _ in range(_NSET)]
        + [pltpu.VMEM((_B, _R, _F), jnp.float32) for _ in range(_NSET)]
        + [pltpu.SemaphoreType.DMA for _ in range(2 * _NSET)]
    ),
)(_sc_body)


def kernel(inputs, positional_encoding):
    b, s, f = inputs.shape
    return _sc_add(inputs, positional_encoding[:s])


# speedup vs baseline: 1.5103x; 1.0296x over previous
import functools

import jax
import jax.numpy as jnp
from jax import lax
from jax.experimental import pallas as pl
from jax.experimental.pallas import tpu as pltpu
from jax.experimental.pallas import tpu_sc as plsc

_B = 4
_S = 8192
_F = 1024
_NC = 2
_NS = 16
_NW = _NC * _NS
_RPW = _S // _NW
_R = 8
_NBLK = _RPW // _R
_NSET = 3


def _sc_body(in_hbm, pe_hbm, out_hbm, *scratch):
    pe_v = list(scratch[0:_NSET])
    in_v = list(scratch[_NSET:2 * _NSET])
    sin = list(scratch[2 * _NSET:3 * _NSET])
    sout = list(scratch[3 * _NSET:4 * _NSET])

    wid = lax.axis_index("s") * _NC + lax.axis_index("c")
    base = wid * _RPW

    def issue_in(i, p):
        r0 = base + i * _R
        return [
            pltpu.async_copy(pe_hbm.at[pl.ds(r0, _R)], pe_v[p], sin[p]),
            pltpu.async_copy(in_hbm.at[:, pl.ds(r0, _R)], in_v[p], sin[p]),
        ]

    def issue_out_half(i, p, half):
        r0 = base + i * _R
        h = _R // 2
        return pltpu.async_copy(
            in_v[p].at[:, pl.ds(half * h, h)],
            out_hbm.at[:, pl.ds(r0 + half * h, h)],
            sout[p])

    hin = [None] * _NSET
    hout = [None] * _NSET
    hin[0] = issue_in(0, 0)
    hin[1] = issue_in(1, 1)

    for i in range(_NBLK):
        p = i % _NSET
        for h in hin[p]:
            h.wait()
        nxt = i + 2
        if nxt < _NBLK:
            p2 = nxt % _NSET
            if hout[p2] is not None:
                for h in hout[p2]:
                    h.wait()
                hout[p2] = None
            hin[p2] = issue_in(nxt, p2)

        houts = []
        for half in range(2):
            def chunk(j, carry, p=p, half=half):
                o = j * 16
                for r in range(half * (_R // 2), (half + 1) * (_R // 2)):
                    pv = pe_v[p][r, pl.ds(o, 16)]
                    for b in range(_B):
                        in_v[p][b, r, pl.ds(o, 16)] = (
                            in_v[p][b, r, pl.ds(o, 16)] + pv
                        )
                return carry

            lax.fori_loop(0, _F // 16, chunk, 0)
            houts.append(issue_out_half(i, p, half))
        hout[p] = houts

    for hs in hout:
        if hs is not None:
            for h in hs:
                h.wait()


_sc_add = functools.partial(
    pl.kernel,
    out_type=jax.ShapeDtypeStruct((_B, _S, _F), jnp.float32),
    mesh=plsc.VectorSubcoreMesh(core_axis_name="c", subcore_axis_name="s"),
    compiler_params=pltpu.CompilerParams(use_tc_tiling_on_sc=True),
    scratch_types=(
        [pltpu.VMEM((_R, _F), jnp.float32) for _ in range(_NSET)]
        + [pltpu.VMEM((_B, _R, _F), jnp.float32) for _ in range(_NSET)]
        + [pltpu.SemaphoreType.DMA for _ in range(2 * _NSET)]
    ),
)(_sc_body)


def kernel(inputs, positional_encoding):
    b, s, f = inputs.shape
    return _sc_add(inputs, positional_encoding[:s])
